# Initial kernel scaffold; baseline (speedup 1.0000x reference)
import jax
import jax.numpy as jnp
from jax.experimental import pallas as pl


def kernel(x, edge_index, W1, b1, W2, b2):
    raise NotImplementedError("write your pallas kernel here")


# speedup vs baseline: 30.6418x; 30.6418x over previous
import functools

import jax
import jax.numpy as jnp
from jax import lax
from jax.experimental import pallas as pl
from jax.experimental.pallas import tpu as pltpu
from jax.experimental.pallas import tpu_sc as plsc

N = 10000
E = 320000
D = 128
H = 16
C = 16

NC = 2
NS = 16
NW = NC * NS
CHUNK = 128
NCH = 79
E_PAD = NW * NCH * CHUNK
ACC_ROWS = 10240
RPT = ACC_ROWS // NS

_mesh = plsc.VectorSubcoreMesh(core_axis_name="c", subcore_axis_name="s")
_sc_params = pltpu.CompilerParams(use_tc_tiling_on_sc=False)


@functools.partial(
    pl.kernel,
    out_type=jax.ShapeDtypeStruct((NC, ACC_ROWS, H), jnp.float32),
    mesh=_mesh,
    scratch_types=[
        pltpu.VMEM((NCH, CHUNK), jnp.int32),
        pltpu.VMEM((CHUNK, H), jnp.float32),
        pltpu.VMEM_SHARED((ACC_ROWS, H), jnp.float32),
    ],
    compiler_params=_sc_params,
)
def _deg_kernel(dst_hbm, ones_hbm, zrows_hbm, out_hbm, dst_v, ones_v, acc_sh):
    cid = lax.axis_index("c")
    sid = lax.axis_index("s")
    wid = sid * NC + cid
    pltpu.sync_copy(zrows_hbm, acc_sh.at[pl.ds(sid * RPT, RPT)])
    pltpu.sync_copy(dst_hbm.at[wid], dst_v)
    pltpu.sync_copy(ones_hbm, ones_v)
    plsc.subcore_barrier()

    def body(j, carry):
        pltpu.sync_copy(ones_v, acc_sh.at[dst_v.at[j]], add=True)
        return carry

    lax.fori_loop(0, NCH, body, 0)
    plsc.subcore_barrier()
    pltpu.sync_copy(acc_sh.at[pl.ds(sid * RPT, RPT)],
                    out_hbm.at[cid, pl.ds(sid * RPT, RPT)])


@functools.partial(
    pl.kernel,
    out_type=jax.ShapeDtypeStruct((NC, ACC_ROWS, H), jnp.float32),
    mesh=_mesh,
    scratch_types=[
        pltpu.VMEM((NCH, CHUNK), jnp.int32),
        pltpu.VMEM((NCH, CHUNK), jnp.int32),
        pltpu.VMEM((CHUNK, H), jnp.float32),
        pltpu.VMEM_SHARED((ACC_ROWS, H), jnp.float32),
        pltpu.SemaphoreType.DMA,
    ],
    compiler_params=_sc_params,
)
def _agg_kernel(g_hbm, src_hbm, dst_hbm, zrows_hbm, out_hbm,
                src_v, dst_v, rows_v, acc_sh, sem):
    cid = lax.axis_index("c")
    sid = lax.axis_index("s")
    wid = sid * NC + cid
    pltpu.sync_copy(zrows_hbm, acc_sh.at[pl.ds(sid * RPT, RPT)])
    pltpu.sync_copy(src_hbm.at[wid], src_v)
    pltpu.sync_copy(dst_hbm.at[wid], dst_v)
    plsc.subcore_barrier()

    def body(j, carry):
        pltpu.async_copy(g_hbm.at[src_v.at[j]], rows_v, sem).wait()
        pltpu.sync_copy(rows_v, acc_sh.at[dst_v.at[j]], add=True)
        return carry

    lax.fori_loop(0, NCH, body, 0)
    plsc.subcore_barrier()
    pltpu.sync_copy(acc_sh.at[pl.ds(sid * RPT, RPT)],
                    out_hbm.at[cid, pl.ds(sid * RPT, RPT)])


def _dense1_body(x_ref, w1_ref, d0_ref, d1_ref, g_ref, dis_ref):
    deg = d0_ref[...] + d1_ref[...] + 1.0
    dis = lax.rsqrt(deg)
    h = jnp.dot(x_ref[...], w1_ref[...], preferred_element_type=jnp.float32)
    g_ref[...] = h * dis
    dis_ref[...] = dis


def _dense2_body(g_ref, a0_ref, a1_ref, dis_ref, b1_ref, w2_ref, out_ref):
    dis = dis_ref[...]
    z = jnp.maximum(dis * (g_ref[...] + a0_ref[...] + a1_ref[...]) + b1_ref[...],
                    0.0)
    out_ref[...] = jnp.dot(z, w2_ref[...], preferred_element_type=jnp.float32) * dis


def _dense3_body(g_ref, a0_ref, a1_ref, dis_ref, b2_ref, out_ref):
    out_ref[...] = (dis_ref[...] * (g_ref[...] + a0_ref[...] + a1_ref[...])
                    + b2_ref[...])


def kernel(x, edge_index, W1, b1, W2, b2):
    src = edge_index[0]
    dst = edge_index[1]
    pad = E_PAD - E
    src3 = jnp.concatenate([src, jnp.zeros((pad,), jnp.int32)]).reshape(
        NW, NCH, CHUNK)
    dst3 = jnp.concatenate([dst, jnp.full((pad,), N, jnp.int32)]).reshape(
        NW, NCH, CHUNK)
    ones2d = jnp.ones((CHUNK, H), jnp.float32)
    zrows = jnp.zeros((RPT, H), jnp.float32)

    degp = _deg_kernel(dst3, ones2d, zrows)
    d0 = degp[0, :N]
    d1 = degp[1, :N]

    g1, dis = pl.pallas_call(
        _dense1_body,
        out_shape=(jax.ShapeDtypeStruct((N, H), jnp.float32),
                   jax.ShapeDtypeStruct((N, H), jnp.float32)),
    )(x, W1, d0, d1)

    a1p = _agg_kernel(g1, src3, dst3, zrows)
    g2 = pl.pallas_call(
        _dense2_body,
        out_shape=jax.ShapeDtypeStruct((N, H), jnp.float32),
    )(g1, a1p[0, :N], a1p[1, :N], dis, b1.reshape(1, H), W2)

    a2p = _agg_kernel(g2, src3, dst3, zrows)
    out = pl.pallas_call(
        _dense3_body,
        out_shape=jax.ShapeDtypeStruct((N, C), jnp.float32),
    )(g2, a2p[0, :N], a2p[1, :N], dis, b2.reshape(1, C))
    return out


# speedup vs baseline: 35.2728x; 1.1511x over previous
import functools

import jax
import jax.numpy as jnp
from jax import lax
from jax.experimental import pallas as pl
from jax.experimental.pallas import tpu as pltpu
from jax.experimental.pallas import tpu_sc as plsc

N = 10000
E = 320000
D = 128
H = 16
C = 16

NC = 2
NS = 16
NW = NC * NS
CHUNK = 128
NCH = 80
NBUF = 4
E_PAD = NW * NCH * CHUNK
ACC_ROWS = 10240
RPT = ACC_ROWS // NS

_mesh = plsc.VectorSubcoreMesh(core_axis_name="c", subcore_axis_name="s")
_sc_params = pltpu.CompilerParams(use_tc_tiling_on_sc=False)


@functools.partial(
    pl.kernel,
    out_type=jax.ShapeDtypeStruct((NC, ACC_ROWS, H), jnp.float32),
    mesh=_mesh,
    scratch_types=[
        pltpu.VMEM((NCH, CHUNK), jnp.int32),
        pltpu.VMEM((CHUNK, H), jnp.float32),
        pltpu.VMEM_SHARED((ACC_ROWS, H), jnp.float32),
        pltpu.SemaphoreType.DMA,
        pltpu.SemaphoreType.DMA,
        pltpu.SemaphoreType.DMA,
        pltpu.SemaphoreType.DMA,
    ],
    compiler_params=_sc_params,
)
def _deg_kernel(dst_hbm, ones_hbm, zrows_hbm, out_hbm, dst_v, ones_v, acc_sh,
                sem0, sem1, sem2, sem3):
    cid = lax.axis_index("c")
    sid = lax.axis_index("s")
    wid = sid * NC + cid
    sems = (sem0, sem1, sem2, sem3)
    pltpu.sync_copy(zrows_hbm, acc_sh.at[pl.ds(sid * RPT, RPT)])
    pltpu.sync_copy(dst_hbm.at[wid], dst_v)
    pltpu.sync_copy(ones_hbm, ones_v)
    plsc.subcore_barrier()

    for b in range(NBUF):
        pltpu.async_copy(ones_v, acc_sh.at[dst_v.at[b]], sems[b], add=True)

    def body(i, carry):
        for b in range(NBUF):
            j = NBUF * i + b
            pltpu.make_async_copy(ones_v, acc_sh.at[dst_v.at[j]],
                                  sems[b]).wait()
            pltpu.async_copy(ones_v, acc_sh.at[dst_v.at[j + NBUF]], sems[b],
                             add=True)
        return carry

    lax.fori_loop(0, NCH // NBUF - 1, body, 0)
    for b in range(NBUF):
        j = NCH - NBUF + b
        pltpu.make_async_copy(ones_v, acc_sh.at[dst_v.at[j]], sems[b]).wait()
    plsc.subcore_barrier()
    pltpu.sync_copy(acc_sh.at[pl.ds(sid * RPT, RPT)],
                    out_hbm.at[cid, pl.ds(sid * RPT, RPT)])


@functools.partial(
    pl.kernel,
    out_type=jax.ShapeDtypeStruct((NC, ACC_ROWS, H), jnp.float32),
    mesh=_mesh,
    scratch_types=[
        pltpu.VMEM((NCH, CHUNK), jnp.int32),
        pltpu.VMEM((NCH, CHUNK), jnp.int32),
        pltpu.VMEM((CHUNK, H), jnp.float32),
        pltpu.VMEM((CHUNK, H), jnp.float32),
        pltpu.VMEM((CHUNK, H), jnp.float32),
        pltpu.VMEM((CHUNK, H), jnp.float32),
        pltpu.VMEM_SHARED((ACC_ROWS, H), jnp.float32),
        pltpu.SemaphoreType.DMA,
        pltpu.SemaphoreType.DMA,
        pltpu.SemaphoreType.DMA,
        pltpu.SemaphoreType.DMA,
    ],
    compiler_params=_sc_params,
)
def _agg_kernel(g_hbm, src_hbm, dst_hbm, zrows_hbm, out_hbm,
                src_v, dst_v, rows0, rows1, rows2, rows3, acc_sh,
                sem0, sem1, sem2, sem3):
    cid = lax.axis_index("c")
    sid = lax.axis_index("s")
    wid = sid * NC + cid
    bufs = (rows0, rows1, rows2, rows3)
    sems = (sem0, sem1, sem2, sem3)
    pltpu.sync_copy(zrows_hbm, acc_sh.at[pl.ds(sid * RPT, RPT)])
    pltpu.sync_copy(src_hbm.at[wid], src_v)
    pltpu.sync_copy(dst_hbm.at[wid], dst_v)
    plsc.subcore_barrier()

    for b in range(NBUF):
        pltpu.async_copy(g_hbm.at[src_v.at[b]], bufs[b], sems[b])

    def body(i, carry):
        for b in range(NBUF):
            j = NBUF * i + b
            pltpu.make_async_copy(g_hbm.at[src_v.at[j]], bufs[b],
                                  sems[b]).wait()
            pltpu.sync_copy(bufs[b], acc_sh.at[dst_v.at[j]], add=True)
            pltpu.async_copy(g_hbm.at[src_v.at[j + NBUF]], bufs[b], sems[b])
        return carry

    lax.fori_loop(0, NCH // NBUF - 1, body, 0)
    for b in range(NBUF):
        j = NCH - NBUF + b
        pltpu.make_async_copy(g_hbm.at[src_v.at[j]], bufs[b], sems[b]).wait()
        pltpu.sync_copy(bufs[b], acc_sh.at[dst_v.at[j]], add=True)
    plsc.subcore_barrier()
    pltpu.sync_copy(acc_sh.at[pl.ds(sid * RPT, RPT)],
                    out_hbm.at[cid, pl.ds(sid * RPT, RPT)])


def _matmul1_body(x_ref, w1_ref, h_ref):
    h_ref[...] = jnp.dot(x_ref[...], w1_ref[...],
                         preferred_element_type=jnp.float32)


def _scale1_body(h_ref, d0_ref, d1_ref, g_ref, dis_ref):
    deg = d0_ref[...] + d1_ref[...] + 1.0
    dis = lax.rsqrt(deg)
    g_ref[...] = h_ref[...] * dis
    dis_ref[...] = dis


def _dense2_body(g_ref, a0_ref, a1_ref, dis_ref, b1_ref, w2_ref, out_ref):
    dis = dis_ref[...]
    z = jnp.maximum(dis * (g_ref[...] + a0_ref[...] + a1_ref[...]) + b1_ref[...],
                    0.0)
    out_ref[...] = jnp.dot(z, w2_ref[...], preferred_element_type=jnp.float32) * dis


def _dense3_body(g_ref, a0_ref, a1_ref, dis_ref, b2_ref, out_ref):
    out_ref[...] = (dis_ref[...] * (g_ref[...] + a0_ref[...] + a1_ref[...])
                    + b2_ref[...])


def kernel(x, edge_index, W1, b1, W2, b2):
    src = edge_index[0]
    dst = edge_index[1]
    pad = E_PAD - E
    src3 = jnp.concatenate([src, jnp.zeros((pad,), jnp.int32)]).reshape(
        NW, NCH, CHUNK)
    dst3 = jnp.concatenate([dst, jnp.full((pad,), N, jnp.int32)]).reshape(
        NW, NCH, CHUNK)
    ones2d = jnp.ones((CHUNK, H), jnp.float32)
    zrows = jnp.zeros((RPT, H), jnp.float32)

    degp = _deg_kernel(dst3, ones2d, zrows)
    d0 = degp[0, :N]
    d1 = degp[1, :N]

    h1 = pl.pallas_call(
        _matmul1_body,
        out_shape=jax.ShapeDtypeStruct((N, H), jnp.float32),
    )(x, W1)
    g1, dis = pl.pallas_call(
        _scale1_body,
        out_shape=(jax.ShapeDtypeStruct((N, H), jnp.float32),
                   jax.ShapeDtypeStruct((N, H), jnp.float32)),
    )(h1, d0, d1)

    a1p = _agg_kernel(g1, src3, dst3, zrows)
    g2 = pl.pallas_call(
        _dense2_body,
        out_shape=jax.ShapeDtypeStruct((N, H), jnp.float32),
    )(g1, a1p[0, :N], a1p[1, :N], dis, b1.reshape(1, H), W2)

    a2p = _agg_kernel(g2, src3, dst3, zrows)
    out = pl.pallas_call(
        _dense3_body,
        out_shape=jax.ShapeDtypeStruct((N, C), jnp.float32),
    )(g2, a2p[0, :N], a2p[1, :N], dis, b2.reshape(1, C))
    return out


# speedup vs baseline: 63.5983x; 1.8030x over previous
import functools

import jax
import jax.numpy as jnp
from jax import lax
from jax.experimental import pallas as pl
from jax.experimental.pallas import tpu as pltpu
from jax.experimental.pallas import tpu_sc as plsc

N = 10000
E = 320000
D = 128
H = 16
C = 16

NC = 2
NS = 16
NW = NC * NS
CHUNK = 125
NCH = 80
NBUF = 8
ACC_ROWS = 10240
RPT = ACC_ROWS // NS

_mesh = plsc.VectorSubcoreMesh(core_axis_name="c", subcore_axis_name="s")
_sc_params = pltpu.CompilerParams(use_tc_tiling_on_sc=False)


@functools.partial(
    pl.kernel,
    out_type=jax.ShapeDtypeStruct((NC, ACC_ROWS, H), jnp.float32),
    mesh=_mesh,
    scratch_types=[
        pltpu.VMEM((NCH, CHUNK), jnp.int32),
        pltpu.VMEM((CHUNK, H), jnp.float32),
        pltpu.VMEM_SHARED((ACC_ROWS, H), jnp.float32),
        [pltpu.SemaphoreType.DMA] * NBUF,
    ],
    compiler_params=_sc_params,
)
def _deg_kernel(ei_hbm, ones_hbm, zrows_hbm, out_hbm, dst_v, ones_v, acc_sh,
                sems):
    cid = lax.axis_index("c")
    sid = lax.axis_index("s")
    wid = sid * NC + cid
    pltpu.sync_copy(zrows_hbm, acc_sh.at[pl.ds(sid * RPT, RPT)])
    pltpu.sync_copy(ei_hbm.at[1, wid], dst_v)
    pltpu.sync_copy(ones_hbm, ones_v)
    plsc.subcore_barrier()

    for b in range(NBUF):
        pltpu.async_copy(ones_v, acc_sh.at[dst_v.at[b]], sems[b], add=True)

    def body(i, carry):
        for b in range(NBUF):
            j = NBUF * i + b
            pltpu.make_async_copy(ones_v, acc_sh.at[dst_v.at[j]],
                                  sems[b]).wait()
            pltpu.async_copy(ones_v, acc_sh.at[dst_v.at[j + NBUF]], sems[b],
                             add=True)
        return carry

    lax.fori_loop(0, NCH // NBUF - 1, body, 0)
    for b in range(NBUF):
        j = NCH - NBUF + b
        pltpu.make_async_copy(ones_v, acc_sh.at[dst_v.at[j]], sems[b]).wait()
    plsc.subcore_barrier()
    pltpu.sync_copy(acc_sh.at[pl.ds(sid * RPT, RPT)],
                    out_hbm.at[cid, pl.ds(sid * RPT, RPT)])


@functools.partial(
    pl.kernel,
    out_type=jax.ShapeDtypeStruct((NC, ACC_ROWS, H), jnp.float32),
    mesh=_mesh,
    scratch_types=[
        pltpu.VMEM((NCH, CHUNK), jnp.int32),
        pltpu.VMEM((NCH, CHUNK), jnp.int32),
        [pltpu.VMEM((CHUNK, H), jnp.float32)] * NBUF,
        pltpu.VMEM_SHARED((ACC_ROWS, H), jnp.float32),
        [pltpu.SemaphoreType.DMA] * NBUF,
    ],
    compiler_params=_sc_params,
)
def _agg_kernel(g_hbm, ei_hbm, zrows_hbm, out_hbm,
                src_v, dst_v, bufs, acc_sh, sems):
    cid = lax.axis_index("c")
    sid = lax.axis_index("s")
    wid = sid * NC + cid
    pltpu.sync_copy(zrows_hbm, acc_sh.at[pl.ds(sid * RPT, RPT)])
    pltpu.sync_copy(ei_hbm.at[0, wid], src_v)
    pltpu.sync_copy(ei_hbm.at[1, wid], dst_v)
    plsc.subcore_barrier()

    for b in range(NBUF):
        pltpu.async_copy(g_hbm.at[src_v.at[b]], bufs[b], sems[b])

    def body(i, carry):
        for b in range(NBUF):
            j = NBUF * i + b
            pltpu.make_async_copy(g_hbm.at[src_v.at[j]], bufs[b],
                                  sems[b]).wait()
            pltpu.sync_copy(bufs[b], acc_sh.at[dst_v.at[j]], add=True)
            pltpu.async_copy(g_hbm.at[src_v.at[j + NBUF]], bufs[b], sems[b])
        return carry

    lax.fori_loop(0, NCH // NBUF - 1, body, 0)
    for b in range(NBUF):
        j = NCH - NBUF + b
        pltpu.make_async_copy(g_hbm.at[src_v.at[j]], bufs[b], sems[b]).wait()
        pltpu.sync_copy(bufs[b], acc_sh.at[dst_v.at[j]], add=True)
    plsc.subcore_barrier()
    pltpu.sync_copy(acc_sh.at[pl.ds(sid * RPT, RPT)],
                    out_hbm.at[cid, pl.ds(sid * RPT, RPT)])


def _matmul1_body(x_ref, w1_ref, h_ref):
    h_ref[...] = jnp.dot(x_ref[...], w1_ref[...],
                         preferred_element_type=jnp.float32)


def _scale1_body(h_ref, dp_ref, g_ref, dis_ref):
    dp = dp_ref[...]
    deg = dp[0, :N] + dp[1, :N] + 1.0
    dis = lax.rsqrt(deg)
    g_ref[...] = h_ref[...] * dis
    dis_ref[...] = dis


def _dense2_body(g_ref, ap_ref, dis_ref, b1_ref, w2_ref, out_ref):
    dis = dis_ref[...]
    ap = ap_ref[...]
    z = jnp.maximum(dis * (g_ref[...] + ap[0, :N] + ap[1, :N]) + b1_ref[...],
                    0.0)
    out_ref[...] = jnp.dot(z, w2_ref[...], preferred_element_type=jnp.float32) * dis


def _dense3_body(g_ref, ap_ref, dis_ref, b2_ref, out_ref):
    ap = ap_ref[...]
    out_ref[...] = (dis_ref[...] * (g_ref[...] + ap[0, :N] + ap[1, :N])
                    + b2_ref[...])


def kernel(x, edge_index, W1, b1, W2, b2):
    ei4 = edge_index.reshape(2, NW, NCH, CHUNK)
    ones2d = jnp.ones((CHUNK, H), jnp.float32)
    zrows = jnp.zeros((RPT, H), jnp.float32)

    degp = _deg_kernel(ei4, ones2d, zrows)

    h1 = pl.pallas_call(
        _matmul1_body,
        out_shape=jax.ShapeDtypeStruct((N, H), jnp.float32),
    )(x, W1)
    g1, dis = pl.pallas_call(
        _scale1_body,
        out_shape=(jax.ShapeDtypeStruct((N, H), jnp.float32),
                   jax.ShapeDtypeStruct((N, H), jnp.float32)),
    )(h1, degp)

    a1p = _agg_kernel(g1, ei4, zrows)
    g2 = pl.pallas_call(
        _dense2_body,
        out_shape=jax.ShapeDtypeStruct((N, H), jnp.float32),
    )(g1, a1p, dis, b1.reshape(1, H), W2)

    a2p = _agg_kernel(g2, ei4, zrows)
    out = pl.pallas_call(
        _dense3_body,
        out_shape=jax.ShapeDtypeStruct((N, C), jnp.float32),
    )(g2, a2p, dis, b2.reshape(1, C))
    return out
